# baseline (speedup 1.0000x reference)
import jax
import jax.numpy as jnp
from jax import lax
from jax.experimental import pallas as pl
from jax.experimental.pallas import tpu as pltpu
from jax.experimental.pallas import tpu_sc as plsc

NUM_NUMERICAL = 13
NUM_EMBED = 26
NUM_FEATS = NUM_NUMERICAL + NUM_EMBED
VOCAB = 100000
EMBED_DIM = 32
BATCH = 4096
OUT_COLS = NUM_NUMERICAL + NUM_EMBED * EMBED_DIM + 1

NW = 32
CHUNK = 64
CHUNKS_PER_WORKER = BATCH // (NW * CHUNK)
SLOTS = 32
GATHER_RUN = 128
N_RUNS = CHUNK * SLOTS // GATHER_RUN


def _body(x_hbm, w_hbm, out_hbm, x_v, idx_v, rows_v, out_v, gsem):
    wid = lax.axis_index("s") * 2 + lax.axis_index("c")
    lane = lax.iota(jnp.int32, 16)

    for k in range(CHUNKS_PER_WORKER):
        b0 = wid * (CHUNK * CHUNKS_PER_WORKER) + k * CHUNK
        pltpu.sync_copy(x_hbm.at[pl.ds(b0, CHUNK), :], x_v)

        def idx_body(b, _):
            rows = jnp.full((16,), b, jnp.int32)
            va = plsc.load_gather(x_v, [rows, lane + NUM_NUMERICAL])
            ga = va.astype(jnp.int32) + lane * VOCAB
            plsc.store_scatter(idx_v, [SLOTS * b + lane], ga)
            m = lane < (NUM_EMBED - 16)
            vb = plsc.load_gather(
                x_v, [rows, lane + (NUM_NUMERICAL + 16)], mask=m
            )
            gb = jnp.where(m, vb.astype(jnp.int32) + (lane + 16) * VOCAB, -1)
            plsc.store_scatter(idx_v, [SLOTS * b + 16 + lane], gb)
            return _

        lax.fori_loop(0, CHUNK, idx_body, None)

        copies = []
        for j in range(N_RUNS):
            copies.append(
                pltpu.async_copy(
                    w_hbm.at[
                        plsc.Indices(
                            idx_v.at[pl.ds(j * GATHER_RUN, GATHER_RUN)],
                            ignored_value=-1,
                        )
                    ],
                    rows_v.at[pl.ds(j * GATHER_RUN, GATHER_RUN), :],
                    gsem,
                )
            )
        for cp in copies:
            cp.wait()

        num_src = jnp.where(lane < NUM_NUMERICAL, lane, NUM_FEATS - 1)
        num_dst = jnp.where(lane < NUM_NUMERICAL, lane, OUT_COLS - 1)
        num_msk = lane < (NUM_NUMERICAL + 1)

        def asm_body(b, _):
            rows = jnp.full((16,), b, jnp.int32)
            v = plsc.load_gather(x_v, [rows, num_src])
            plsc.store_scatter(out_v, [rows, num_dst], v, mask=num_msk)
            for t in range(NUM_EMBED):
                r = jnp.full((16,), SLOTS * b + t, jnp.int32)
                for h in range(2):
                    ve = plsc.load_gather(rows_v, [r, lane + 16 * h])
                    plsc.store_scatter(
                        out_v,
                        [rows, lane + (NUM_NUMERICAL + EMBED_DIM * t + 16 * h)],
                        ve,
                    )
            return _

        lax.fori_loop(0, CHUNK, asm_body, None)

        pltpu.sync_copy(out_v, out_hbm.at[pl.ds(b0, CHUNK), :])


def kernel(x, W):
    w_flat = W.reshape(NUM_EMBED * VOCAB, EMBED_DIM)
    mesh = plsc.VectorSubcoreMesh(core_axis_name="c", subcore_axis_name="s")
    f = pl.kernel(
        _body,
        out_type=jax.ShapeDtypeStruct((BATCH, OUT_COLS), jnp.float32),
        mesh=mesh,
        scratch_types=[
            pltpu.VMEM((CHUNK, NUM_FEATS), jnp.float32),
            pltpu.VMEM((CHUNK * SLOTS,), jnp.int32),
            pltpu.VMEM((CHUNK * SLOTS, EMBED_DIM), jnp.float32),
            pltpu.VMEM((CHUNK, OUT_COLS), jnp.float32),
            pltpu.SemaphoreType.DMA,
        ],
        compiler_params=pltpu.CompilerParams(
            use_tc_tiling_on_sc=False, needs_layout_passes=False
        ),
    )
    return f(x, w_flat)


# speedup vs baseline: 1.0017x; 1.0017x over previous
import jax
import jax.numpy as jnp
from jax import lax
from jax.experimental import pallas as pl
from jax.experimental.pallas import tpu as pltpu
from jax.experimental.pallas import tpu_sc as plsc

NUM_NUMERICAL = 13
NUM_EMBED = 26
NUM_FEATS = NUM_NUMERICAL + NUM_EMBED
VOCAB = 100000
EMBED_DIM = 32
BATCH = 4096
OUT_COLS = NUM_NUMERICAL + NUM_EMBED * EMBED_DIM + 1

NW = 32
CHUNK = 64
CHUNKS_PER_WORKER = BATCH // (NW * CHUNK)


def _body(x_hbm, w_hbm, out_hbm, x_v, idx_v, rows_v, out_v, gsem):
    wid = lax.axis_index("s") * 2 + lax.axis_index("c")
    lane = lax.iota(jnp.int32, 16)

    for k in range(CHUNKS_PER_WORKER):
        b0 = wid * (CHUNK * CHUNKS_PER_WORKER) + k * CHUNK
        pltpu.sync_copy(x_hbm.at[pl.ds(b0, CHUNK), :], x_v)

        def idx_body(b, _):
            rows = jnp.full((16,), b, jnp.int32)
            va = plsc.load_gather(x_v, [rows, lane + NUM_NUMERICAL])
            ga = va.astype(jnp.int32)
            plsc.store_scatter(idx_v, [lane * CHUNK + b], ga)
            m = lane < (NUM_EMBED - 16)
            vb = plsc.load_gather(
                x_v, [rows, lane + (NUM_NUMERICAL + 16)], mask=m
            )
            gb = vb.astype(jnp.int32)
            plsc.store_scatter(idx_v, [(lane + 16) * CHUNK + b], gb, mask=m)
            return _

        lax.fori_loop(0, CHUNK, idx_body, None)

        copies = []
        for t in range(NUM_EMBED):
            copies.append(
                pltpu.async_copy(
                    w_hbm.at[t].at[idx_v.at[pl.ds(t * CHUNK, CHUNK)]],
                    rows_v.at[pl.ds(t * CHUNK, CHUNK), :],
                    gsem,
                )
            )
        for cp in copies:
            cp.wait()

        num_src = jnp.where(lane < NUM_NUMERICAL, lane, NUM_FEATS - 1)
        num_dst = jnp.where(lane < NUM_NUMERICAL, lane, OUT_COLS - 1)
        num_msk = lane < (NUM_NUMERICAL + 1)

        def asm_body(b, _):
            rows = jnp.full((16,), b, jnp.int32)
            v = plsc.load_gather(x_v, [rows, num_src])
            plsc.store_scatter(out_v, [rows, num_dst], v, mask=num_msk)
            for t in range(NUM_EMBED):
                r = jnp.full((16,), t * CHUNK + b, jnp.int32)
                for h in range(2):
                    ve = plsc.load_gather(rows_v, [r, lane + 16 * h])
                    plsc.store_scatter(
                        out_v,
                        [rows, lane + (NUM_NUMERICAL + EMBED_DIM * t + 16 * h)],
                        ve,
                    )
            return _

        lax.fori_loop(0, CHUNK, asm_body, None)

        pltpu.sync_copy(out_v, out_hbm.at[pl.ds(b0, CHUNK), :])


def kernel(x, W):
    mesh = plsc.VectorSubcoreMesh(core_axis_name="c", subcore_axis_name="s")
    f = pl.kernel(
        _body,
        out_type=jax.ShapeDtypeStruct((BATCH, OUT_COLS), jnp.float32),
        mesh=mesh,
        scratch_types=[
            pltpu.VMEM((CHUNK, NUM_FEATS), jnp.float32),
            pltpu.VMEM((CHUNK * NUM_EMBED,), jnp.int32),
            pltpu.VMEM((CHUNK * NUM_EMBED, EMBED_DIM), jnp.float32),
            pltpu.VMEM((CHUNK, OUT_COLS), jnp.float32),
            pltpu.SemaphoreType.DMA,
        ],
        compiler_params=pltpu.CompilerParams(
            use_tc_tiling_on_sc=False, needs_layout_passes=False
        ),
    )
    return f(x, W)
